# baseline (speedup 1.0000x reference)
import functools

import jax
import jax.numpy as jnp
from jax import lax
from jax.experimental import pallas as pl
from jax.experimental.pallas import tpu as pltpu
from jax.experimental.pallas import tpu_sc as plsc

_E = 8
_H = 2048
_I = 1024
_EPAD = 128
_T = 2048
_PAIRS = 2 * _T
_BM = 256
_NP = _PAIRS + _E * _BM
_NB = _NP // _BM
_NSC = 32


def _router_body(x_ref, wg_ref, ep_ref, wp_ref):
    logits = jnp.dot(x_ref[...], wg_ref[...], preferred_element_type=jnp.float32)
    col = jax.lax.broadcasted_iota(jnp.int32, logits.shape, 1)
    masked = jnp.where(col < _E, logits, jnp.float32(-1e30))
    m = jnp.max(masked, axis=1, keepdims=True)
    p = jnp.exp(masked - m)
    p = p / jnp.sum(p, axis=1, keepdims=True)
    w1v = jnp.max(p, axis=1, keepdims=True)
    i1 = jnp.argmax(p, axis=1)[:, None]
    p2 = jnp.where(col == i1, jnp.float32(-1.0), p)
    w2v = jnp.max(p2, axis=1, keepdims=True)
    i2 = jnp.argmax(p2, axis=1)[:, None]
    s = w1v + w2v
    ep_ref[...] = (jnp.where(col == 0, i1, 0)
                   + jnp.where(col == 1, i2, 0)).astype(jnp.int32)
    wp_ref[...] = jnp.where(col == 0, w1v / s, 0.0) + jnp.where(
        col == 1, w2v / s, 0.0)


def _assign_body(e32_ref, pos_ref, meta_ref):
    ef = e32_ref[...]
    ri = jax.lax.broadcasted_iota(jnp.int32, (128, 128), 0)
    ci = jax.lax.broadcasted_iota(jnp.int32, (128, 128), 1)
    tri128 = (ri < ci).astype(jnp.float32)
    ones128 = jnp.ones((128, 128), jnp.float32)
    r32 = jax.lax.broadcasted_iota(jnp.int32, (32, 32), 0)
    c32 = jax.lax.broadcasted_iota(jnp.int32, (32, 32), 1)
    tri32 = (c32 < r32).astype(jnp.float32)
    mrow = jax.lax.broadcasted_iota(jnp.int32, (8, 128), 0)
    mcol = jax.lax.broadcasted_iota(jnp.int32, (8, 128), 1)

    pos = jnp.zeros((32, 128), jnp.float32)
    blk = jnp.zeros((8, 128), jnp.int32)
    off = jnp.float32(0.0)
    laste = jnp.int32(0)
    for e in range(_E):
        mask = (ef == e).astype(jnp.float32)
        intra = jnp.dot(mask, tri128, preferred_element_type=jnp.float32)
        srow_b = jnp.dot(mask, ones128, preferred_element_type=jnp.float32)
        rowpref = jnp.dot(tri32, srow_b, preferred_element_type=jnp.float32)
        rank = intra + rowpref
        cnt = jnp.sum(mask)
        nb = jnp.ceil(cnt / _BM)
        pos = pos + mask * (off + rank)
        start = (off / _BM).astype(jnp.int32)
        nbi = nb.astype(jnp.int32)
        act = (mcol >= start) & (mcol < start + nbi)
        blk = (blk + jnp.where((mrow == 0) & act, e, 0)
               + jnp.where((mrow == 1) & act, 1, 0))
        laste = jnp.where(nbi > 0, jnp.int32(e), laste)
        off = off + nb * _BM
    total = (off / _BM).astype(jnp.int32)
    blk = blk + jnp.where((mrow == 0) & (mcol >= total), laste, 0)
    pos_ref[...] = pos.astype(jnp.int32)
    meta_ref[...] = blk


def _scatter_body(pos_hbm, st_hbm, pos_v, st_v):
    c = lax.axis_index("c")
    s = lax.axis_index("s")

    @pl.when((c == 0) & (s == 0))
    def _():
        pltpu.sync_copy(pos_hbm, pos_v)

        def zero(j, carry):
            st_v[pl.ds(j * 16, 16)] = jnp.zeros((16,), jnp.int32)
            return carry

        lax.fori_loop(0, _NP // 16, zero, 0)
        iota = lax.iota(jnp.int32, 16)

        def scat(j, carry):
            idx = pos_v[pl.ds(j * 16, 16)]
            tok = lax.shift_right_logical(j * 16 + iota, 1)
            plsc.store_scatter(st_v, [idx], tok)
            return carry

        lax.fori_loop(0, _PAIRS // 16, scat, 0)
        pltpu.sync_copy(st_v, st_hbm)


def _gatherx_body(st_hbm, x_hbm, xs_hbm, idx_v, buf, sem):
    c = lax.axis_index("c")
    s = lax.axis_index("s")
    wid = s * 2 + c
    rows = _NP // _NSC
    base = wid * rows

    def chunk(j, carry):
        pltpu.sync_copy(st_hbm.at[pl.ds(base + j * 16, 16)], idx_v)
        pltpu.async_copy(x_hbm.at[idx_v], buf, sem).wait()
        pltpu.sync_copy(buf, xs_hbm.at[pl.ds(base + j * 16, 16)])
        return carry

    lax.fori_loop(0, rows // 16, chunk, 0)


def _gemm_body(meta_ref, xs_ref, w1_ref, w3_ref, w2_ref, o_ref):
    i = pl.program_id(1)
    b = pl.program_id(0)
    valid = meta_ref[_NB + b]

    @pl.when(valid == 1)
    def _():
        xs = xs_ref[...]
        g = jnp.dot(xs, w1_ref[0], preferred_element_type=jnp.float32)
        u = jnp.dot(xs, w3_ref[0], preferred_element_type=jnp.float32)
        h = (g * jax.nn.sigmoid(g)) * u
        o = jnp.dot(h, w2_ref[0], preferred_element_type=jnp.float32)

        @pl.when(i == 0)
        def _():
            o_ref[...] = o

        @pl.when(i > 0)
        def _():
            o_ref[...] += o


def _combine_body(pos_hbm, w_hbm, o_hbm, out_hbm, pos_v, w_v, idxa_v, idxb_v,
                  bufa, bufb, outb, sema, semb):
    c = lax.axis_index("c")
    s = lax.axis_index("s")
    wid = s * 2 + c
    tok_per = _T // _NSC
    base_t = wid * tok_per
    pltpu.sync_copy(pos_hbm.at[pl.ds(2 * base_t, 2 * tok_per)], pos_v)
    pltpu.sync_copy(w_hbm.at[pl.ds(2 * base_t, 2 * tok_per)], w_v)
    iota = lax.iota(jnp.int32, 16)

    def chunk(cc, carry):
        lpair = cc * 32 + 2 * iota
        idxa_v[...] = plsc.load_gather(pos_v, [lpair])
        idxb_v[...] = plsc.load_gather(pos_v, [lpair + 1])
        cpa = pltpu.async_copy(o_hbm.at[idxa_v], bufa, sema)
        cpb = pltpu.async_copy(o_hbm.at[idxb_v], bufb, semb)
        cpa.wait()
        cpb.wait()

        def row(r, carry2):
            wa = plsc.load_gather(w_v, [cc * 32 + 2 * r + jnp.zeros((16,), jnp.int32)])
            wb = plsc.load_gather(w_v, [cc * 32 + 2 * r + 1 + jnp.zeros((16,), jnp.int32)])

            def vec(v, carry3):
                outb[r, pl.ds(v * 16, 16)] = (
                    wa * bufa[r, pl.ds(v * 16, 16)]
                    + wb * bufb[r, pl.ds(v * 16, 16)])
                return carry3

            lax.fori_loop(0, _H // 16, vec, 0)
            return carry2

        lax.fori_loop(0, 16, row, 0)
        pltpu.sync_copy(outb, out_hbm.at[pl.ds(base_t + cc * 16, 16)])
        return carry

    lax.fori_loop(0, tok_per // 16, chunk, 0)


def kernel(hidden_states, W_gate, w1, w3, w2):
    orig_shape = hidden_states.shape
    x = hidden_states.reshape(-1, _H)
    t = x.shape[0]

    wg_pad = jnp.zeros((_H, _EPAD), jnp.float32).at[:, :_E].set(W_gate)

    bt_r = 512
    epair, wpair = pl.pallas_call(
        _router_body,
        grid=(t // bt_r,),
        in_specs=[
            pl.BlockSpec((bt_r, _H), lambda i: (i, 0)),
            pl.BlockSpec((_H, _EPAD), lambda i: (0, 0)),
        ],
        out_specs=[
            pl.BlockSpec((bt_r, _EPAD), lambda i: (i, 0)),
            pl.BlockSpec((bt_r, _EPAD), lambda i: (i, 0)),
        ],
        out_shape=[
            jax.ShapeDtypeStruct((t, _EPAD), jnp.int32),
            jax.ShapeDtypeStruct((t, _EPAD), jnp.float32),
        ],
    )(x, wg_pad)

    e_flat = epair[:, :2].reshape(_PAIRS)
    w_flat = wpair[:, :2].reshape(_PAIRS)

    pos32, meta = pl.pallas_call(
        _assign_body,
        in_specs=[pl.BlockSpec((32, 128), lambda: (0, 0))],
        out_specs=[
            pl.BlockSpec((32, 128), lambda: (0, 0)),
            pl.BlockSpec((8, 128), lambda: (0, 0)),
        ],
        out_shape=[
            jax.ShapeDtypeStruct((32, 128), jnp.int32),
            jax.ShapeDtypeStruct((8, 128), jnp.int32),
        ],
    )(e_flat.reshape(32, 128))

    pos_flat = pos32.reshape(_PAIRS)
    meta_vec = jnp.concatenate([meta[0, :_NB], meta[1, :_NB]])

    mesh = plsc.VectorSubcoreMesh(core_axis_name="c", subcore_axis_name="s")

    sorted_tok = pl.kernel(
        _scatter_body,
        out_type=jax.ShapeDtypeStruct((_NP,), jnp.int32),
        mesh=mesh,
        compiler_params=pltpu.CompilerParams(needs_layout_passes=False),
        scratch_types=[
            pltpu.VMEM((_PAIRS,), jnp.int32),
            pltpu.VMEM((_NP,), jnp.int32),
        ],
    )(pos_flat)

    x_sorted = pl.kernel(
        _gatherx_body,
        out_type=jax.ShapeDtypeStruct((_NP, _H), jnp.float32),
        mesh=mesh,
        scratch_types=[
            pltpu.VMEM((16,), jnp.int32),
            pltpu.VMEM((16, _H), jnp.float32),
            pltpu.SemaphoreType.DMA,
        ],
    )(sorted_tok, x)

    ti = 512
    o_sorted = pl.pallas_call(
        _gemm_body,
        grid_spec=pltpu.PrefetchScalarGridSpec(
            num_scalar_prefetch=1,
            grid=(_NB, _I // ti),
            in_specs=[
                pl.BlockSpec((_BM, _H), lambda b, i, m: (b, 0)),
                pl.BlockSpec((1, _H, ti), lambda b, i, m: (m[b], 0, i)),
                pl.BlockSpec((1, _H, ti), lambda b, i, m: (m[b], 0, i)),
                pl.BlockSpec((1, ti, _H), lambda b, i, m: (m[b], i, 0)),
            ],
            out_specs=pl.BlockSpec((_BM, _H), lambda b, i, m: (b, 0)),
        ),
        out_shape=jax.ShapeDtypeStruct((_NP, _H), jnp.float32),
        compiler_params=pltpu.CompilerParams(
            dimension_semantics=("arbitrary", "arbitrary")),
    )(meta_vec, x_sorted, w1, w3, w2)

    out = pl.kernel(
        _combine_body,
        out_type=jax.ShapeDtypeStruct((t, _H), jnp.float32),
        mesh=mesh,
        compiler_params=pltpu.CompilerParams(needs_layout_passes=False),
        scratch_types=[
            pltpu.VMEM((2 * _T // _NSC,), jnp.int32),
            pltpu.VMEM((2 * _T // _NSC,), jnp.float32),
            pltpu.VMEM((16,), jnp.int32),
            pltpu.VMEM((16,), jnp.int32),
            pltpu.VMEM((16, _H), jnp.float32),
            pltpu.VMEM((16, _H), jnp.float32),
            pltpu.VMEM((16, _H), jnp.float32),
            pltpu.SemaphoreType.DMA,
            pltpu.SemaphoreType.DMA,
        ],
    )(pos_flat, w_flat, o_sorted)

    return out.reshape(orig_shape)


# speedup vs baseline: 1.0697x; 1.0697x over previous
import functools

import jax
import jax.numpy as jnp
from jax import lax
from jax.experimental import pallas as pl
from jax.experimental.pallas import tpu as pltpu
from jax.experimental.pallas import tpu_sc as plsc

_E = 8
_H = 2048
_I = 1024
_EPAD = 128
_T = 2048
_PAIRS = 2 * _T
_BM = 256
_NP = _PAIRS + _E * _BM
_NB = _NP // _BM
_NSC = 32


def _router_body(x_ref, wg_ref, ep_ref, wp_ref):
    logits = jnp.dot(x_ref[...], wg_ref[...], preferred_element_type=jnp.float32)
    col = jax.lax.broadcasted_iota(jnp.int32, logits.shape, 1)
    masked = jnp.where(col < _E, logits, jnp.float32(-1e30))
    m = jnp.max(masked, axis=1, keepdims=True)
    p = jnp.exp(masked - m)
    p = p / jnp.sum(p, axis=1, keepdims=True)
    w1v = jnp.max(p, axis=1, keepdims=True)
    i1 = jnp.argmax(p, axis=1)[:, None]
    p2 = jnp.where(col == i1, jnp.float32(-1.0), p)
    w2v = jnp.max(p2, axis=1, keepdims=True)
    i2 = jnp.argmax(p2, axis=1)[:, None]
    s = w1v + w2v
    ep_ref[...] = (jnp.where(col == 0, i1, 0)
                   + jnp.where(col == 1, i2, 0)).astype(jnp.int32)
    wp_ref[...] = jnp.where(col == 0, w1v / s, 0.0) + jnp.where(
        col == 1, w2v / s, 0.0)


def _assign_body(e32_ref, pos_ref, meta_ref):
    ef = e32_ref[...]
    ri = jax.lax.broadcasted_iota(jnp.int32, (128, 128), 0)
    ci = jax.lax.broadcasted_iota(jnp.int32, (128, 128), 1)
    tri128 = (ri < ci).astype(jnp.float32)
    ones128 = jnp.ones((128, 128), jnp.float32)
    r32 = jax.lax.broadcasted_iota(jnp.int32, (32, 32), 0)
    c32 = jax.lax.broadcasted_iota(jnp.int32, (32, 32), 1)
    tri32 = (c32 < r32).astype(jnp.float32)
    mrow = jax.lax.broadcasted_iota(jnp.int32, (8, 128), 0)
    mcol = jax.lax.broadcasted_iota(jnp.int32, (8, 128), 1)

    pos = jnp.zeros((32, 128), jnp.float32)
    blk = jnp.zeros((8, 128), jnp.int32)
    off = jnp.float32(0.0)
    laste = jnp.int32(0)
    for e in range(_E):
        mask = (ef == e).astype(jnp.float32)
        intra = jnp.dot(mask, tri128, preferred_element_type=jnp.float32)
        srow_b = jnp.dot(mask, ones128, preferred_element_type=jnp.float32)
        rowpref = jnp.dot(tri32, srow_b, preferred_element_type=jnp.float32)
        rank = intra + rowpref
        cnt = jnp.sum(mask)
        nb = jnp.ceil(cnt / _BM)
        pos = pos + mask * (off + rank)
        start = (off / _BM).astype(jnp.int32)
        nbi = nb.astype(jnp.int32)
        act = (mcol >= start) & (mcol < start + nbi)
        blk = (blk + jnp.where((mrow == 0) & act, e, 0)
               + jnp.where((mrow == 1) & act, 1, 0))
        laste = jnp.where(nbi > 0, jnp.int32(e), laste)
        off = off + nb * _BM
    total = (off / _BM).astype(jnp.int32)
    blk = blk + jnp.where((mrow == 0) & (mcol >= total), laste, 0)
    pos_ref[...] = pos.astype(jnp.int32)
    meta_ref[...] = blk


def _scatter_body(pos_hbm, st_hbm, pos_v, st_v):
    c = lax.axis_index("c")
    s = lax.axis_index("s")

    @pl.when((c == 0) & (s == 0))
    def _():
        pltpu.sync_copy(pos_hbm, pos_v)

        def zero(j, carry):
            st_v[pl.ds(j * 16, 16)] = jnp.zeros((16,), jnp.int32)
            return carry

        lax.fori_loop(0, _NP // 16, zero, 0)
        iota = lax.iota(jnp.int32, 16)

        def scat(j, carry):
            idx = pos_v[pl.ds(j * 16, 16)]
            tok = lax.shift_right_logical(j * 16 + iota, 1)
            plsc.store_scatter(st_v, [idx], tok)
            return carry

        lax.fori_loop(0, _PAIRS // 16, scat, 0)
        pltpu.sync_copy(st_v, st_hbm)


def _gatherx_body(st_hbm, x_hbm, xs_hbm, idx_v, buf, gs0, gs1, gs2, ws0,
                  ws1, ws2):
    c = lax.axis_index("c")
    s = lax.axis_index("s")
    wid = s * 2 + c
    rows = _NP // _NSC
    base = wid * rows
    nch = rows // 16
    pltpu.sync_copy(st_hbm.at[pl.ds(base, rows)], idx_v)
    gsem = (gs0, gs1, gs2)
    wsem = (ws0, ws1, ws2)
    cg = [None] * nch
    cw = [None] * nch
    for j in range(nch):
        if j >= 3:
            cw[j - 3].wait()
        cg[j] = pltpu.async_copy(
            x_hbm.at[idx_v.at[pl.ds(j * 16, 16)]], buf.at[j % 3], gsem[j % 3])
        if j >= 2:
            cg[j - 2].wait()
            cw[j - 2] = pltpu.async_copy(
                buf.at[(j - 2) % 3], xs_hbm.at[pl.ds(base + (j - 2) * 16, 16)],
                wsem[(j - 2) % 3])
    for j in range(nch - 2, nch):
        cg[j].wait()
        cw[j] = pltpu.async_copy(
            buf.at[j % 3], xs_hbm.at[pl.ds(base + j * 16, 16)], wsem[j % 3])
    for j in range(nch - 3, nch):
        cw[j].wait()


def _gemm_body(meta_ref, xs_ref, w1_ref, w3_ref, w2_ref, o_ref):
    b = pl.program_id(0)
    valid = meta_ref[_NB + b]

    @pl.when(valid == 1)
    def _():
        xs = xs_ref[...]
        g = jnp.dot(xs, w1_ref[0], preferred_element_type=jnp.float32)
        u = jnp.dot(xs, w3_ref[0], preferred_element_type=jnp.float32)
        h = (g * jax.nn.sigmoid(g)) * u
        o_ref[...] = jnp.dot(h, w2_ref[0], preferred_element_type=jnp.float32)


def _combine_body(pos_hbm, w_hbm, o_hbm, out_hbm, pos_v, w_v, idxa_v, idxb_v,
                  bufa, bufb, outb, sema, semb):
    c = lax.axis_index("c")
    s = lax.axis_index("s")
    wid = s * 2 + c
    tok_per = _T // _NSC
    base_t = wid * tok_per
    pltpu.sync_copy(pos_hbm.at[pl.ds(2 * base_t, 2 * tok_per)], pos_v)
    pltpu.sync_copy(w_hbm.at[pl.ds(2 * base_t, 2 * tok_per)], w_v)
    iota = lax.iota(jnp.int32, 16)

    def chunk(cc, carry):
        lpair = cc * 32 + 2 * iota
        idxa_v[...] = plsc.load_gather(pos_v, [lpair])
        idxb_v[...] = plsc.load_gather(pos_v, [lpair + 1])
        cpa = pltpu.async_copy(o_hbm.at[idxa_v], bufa, sema)
        cpb = pltpu.async_copy(o_hbm.at[idxb_v], bufb, semb)
        cpa.wait()
        cpb.wait()

        def row(r, carry2):
            wa = plsc.load_gather(w_v, [cc * 32 + 2 * r + jnp.zeros((16,), jnp.int32)])
            wb = plsc.load_gather(w_v, [cc * 32 + 2 * r + 1 + jnp.zeros((16,), jnp.int32)])

            def vec(v, carry3):
                outb[r, pl.ds(v * 16, 16)] = (
                    wa * bufa[r, pl.ds(v * 16, 16)]
                    + wb * bufb[r, pl.ds(v * 16, 16)])
                return carry3

            lax.fori_loop(0, _H // 16, vec, 0)
            return carry2

        lax.fori_loop(0, 16, row, 0)
        pltpu.sync_copy(outb, out_hbm.at[pl.ds(base_t + cc * 16, 16)])
        return carry

    lax.fori_loop(0, tok_per // 16, chunk, 0)


def kernel(hidden_states, W_gate, w1, w3, w2):
    orig_shape = hidden_states.shape
    x = hidden_states.reshape(-1, _H)
    t = x.shape[0]

    wg_pad = jnp.zeros((_H, _EPAD), jnp.float32).at[:, :_E].set(W_gate)

    bt_r = 512
    epair, wpair = pl.pallas_call(
        _router_body,
        grid=(t // bt_r,),
        in_specs=[
            pl.BlockSpec((bt_r, _H), lambda i: (i, 0)),
            pl.BlockSpec((_H, _EPAD), lambda i: (0, 0)),
        ],
        out_specs=[
            pl.BlockSpec((bt_r, _EPAD), lambda i: (i, 0)),
            pl.BlockSpec((bt_r, _EPAD), lambda i: (i, 0)),
        ],
        out_shape=[
            jax.ShapeDtypeStruct((t, _EPAD), jnp.int32),
            jax.ShapeDtypeStruct((t, _EPAD), jnp.float32),
        ],
    )(x, wg_pad)

    e_flat = epair[:, :2].reshape(_PAIRS)
    w_flat = wpair[:, :2].reshape(_PAIRS)

    pos32, meta = pl.pallas_call(
        _assign_body,
        in_specs=[pl.BlockSpec((32, 128), lambda: (0, 0))],
        out_specs=[
            pl.BlockSpec((32, 128), lambda: (0, 0)),
            pl.BlockSpec((8, 128), lambda: (0, 0)),
        ],
        out_shape=[
            jax.ShapeDtypeStruct((32, 128), jnp.int32),
            jax.ShapeDtypeStruct((8, 128), jnp.int32),
        ],
    )(e_flat.reshape(32, 128))

    pos_flat = pos32.reshape(_PAIRS)
    meta_vec = jnp.concatenate([meta[0, :_NB], meta[1, :_NB]])

    mesh = plsc.VectorSubcoreMesh(core_axis_name="c", subcore_axis_name="s")

    sorted_tok = pl.kernel(
        _scatter_body,
        out_type=jax.ShapeDtypeStruct((_NP,), jnp.int32),
        mesh=mesh,
        compiler_params=pltpu.CompilerParams(needs_layout_passes=False),
        scratch_types=[
            pltpu.VMEM((_PAIRS,), jnp.int32),
            pltpu.VMEM((_NP,), jnp.int32),
        ],
    )(pos_flat)

    x_sorted = pl.kernel(
        _gatherx_body,
        out_type=jax.ShapeDtypeStruct((_NP, _H), jnp.float32),
        mesh=mesh,
        scratch_types=[
            pltpu.VMEM((_NP // _NSC,), jnp.int32),
            pltpu.VMEM((3, 16, _H), jnp.float32),
            pltpu.SemaphoreType.DMA,
            pltpu.SemaphoreType.DMA,
            pltpu.SemaphoreType.DMA,
            pltpu.SemaphoreType.DMA,
            pltpu.SemaphoreType.DMA,
            pltpu.SemaphoreType.DMA,
        ],
    )(sorted_tok, x)

    o_sorted = pl.pallas_call(
        _gemm_body,
        grid_spec=pltpu.PrefetchScalarGridSpec(
            num_scalar_prefetch=1,
            grid=(_NB,),
            in_specs=[
                pl.BlockSpec((_BM, _H), lambda b, m: (b, 0)),
                pl.BlockSpec((1, _H, _I), lambda b, m: (m[b], 0, 0)),
                pl.BlockSpec((1, _H, _I), lambda b, m: (m[b], 0, 0)),
                pl.BlockSpec((1, _I, _H), lambda b, m: (m[b], 0, 0)),
            ],
            out_specs=pl.BlockSpec((_BM, _H), lambda b, m: (b, 0)),
        ),
        out_shape=jax.ShapeDtypeStruct((_NP, _H), jnp.float32),
        compiler_params=pltpu.CompilerParams(
            dimension_semantics=("arbitrary",)),
    )(meta_vec, x_sorted, w1, w3, w2)

    out = pl.kernel(
        _combine_body,
        out_type=jax.ShapeDtypeStruct((t, _H), jnp.float32),
        mesh=mesh,
        compiler_params=pltpu.CompilerParams(needs_layout_passes=False),
        scratch_types=[
            pltpu.VMEM((2 * _T // _NSC,), jnp.int32),
            pltpu.VMEM((2 * _T // _NSC,), jnp.float32),
            pltpu.VMEM((16,), jnp.int32),
            pltpu.VMEM((16,), jnp.int32),
            pltpu.VMEM((16, _H), jnp.float32),
            pltpu.VMEM((16, _H), jnp.float32),
            pltpu.VMEM((16, _H), jnp.float32),
            pltpu.SemaphoreType.DMA,
            pltpu.SemaphoreType.DMA,
        ],
    )(pos_flat, w_flat, o_sorted)

    return out.reshape(orig_shape)


# speedup vs baseline: 1.1149x; 1.0422x over previous
import functools

import jax
import jax.numpy as jnp
from jax import lax
from jax.experimental import pallas as pl
from jax.experimental.pallas import tpu as pltpu
from jax.experimental.pallas import tpu_sc as plsc

_E = 8
_H = 2048
_I = 1024
_EPAD = 128
_T = 2048
_PAIRS = 2 * _T
_BM = 256
_NP = _PAIRS + _E * _BM
_NB = _NP // _BM
_NSC = 32


def _router_body(x_ref, wg_ref, ep_ref, wp_ref):
    logits = jnp.dot(x_ref[...], wg_ref[...], preferred_element_type=jnp.float32)
    col = jax.lax.broadcasted_iota(jnp.int32, logits.shape, 1)
    masked = jnp.where(col < _E, logits, jnp.float32(-1e30))
    m = jnp.max(masked, axis=1, keepdims=True)
    p = jnp.exp(masked - m)
    p = p / jnp.sum(p, axis=1, keepdims=True)
    w1v = jnp.max(p, axis=1, keepdims=True)
    i1 = jnp.argmax(p, axis=1)[:, None]
    p2 = jnp.where(col == i1, jnp.float32(-1.0), p)
    w2v = jnp.max(p2, axis=1, keepdims=True)
    i2 = jnp.argmax(p2, axis=1)[:, None]
    s = w1v + w2v
    ep_ref[...] = (jnp.where(col == 0, i1, 0)
                   + jnp.where(col == 1, i2, 0)).astype(jnp.int32)
    wp_ref[...] = jnp.where(col == 0, w1v / s, 0.0) + jnp.where(
        col == 1, w2v / s, 0.0)


def _assign_body(e32_ref, pos_ref, meta_ref):
    ef = e32_ref[...]
    ri = jax.lax.broadcasted_iota(jnp.int32, (128, 128), 0)
    ci = jax.lax.broadcasted_iota(jnp.int32, (128, 128), 1)
    tri128 = (ri < ci).astype(jnp.float32)
    ones128 = jnp.ones((128, 128), jnp.float32)
    r32 = jax.lax.broadcasted_iota(jnp.int32, (32, 32), 0)
    c32 = jax.lax.broadcasted_iota(jnp.int32, (32, 32), 1)
    tri32 = (c32 < r32).astype(jnp.float32)
    mrow = jax.lax.broadcasted_iota(jnp.int32, (8, 128), 0)
    mcol = jax.lax.broadcasted_iota(jnp.int32, (8, 128), 1)

    pos = jnp.zeros((32, 128), jnp.float32)
    blk = jnp.zeros((8, 128), jnp.int32)
    off = jnp.float32(0.0)
    laste = jnp.int32(0)
    for e in range(_E):
        mask = (ef == e).astype(jnp.float32)
        intra = jnp.dot(mask, tri128, preferred_element_type=jnp.float32)
        srow_b = jnp.dot(mask, ones128, preferred_element_type=jnp.float32)
        rowpref = jnp.dot(tri32, srow_b, preferred_element_type=jnp.float32)
        rank = intra + rowpref
        cnt = jnp.sum(mask)
        nb = jnp.ceil(cnt / _BM)
        pos = pos + mask * (off + rank)
        start = (off / _BM).astype(jnp.int32)
        nbi = nb.astype(jnp.int32)
        act = (mcol >= start) & (mcol < start + nbi)
        blk = (blk + jnp.where((mrow == 0) & act, e, 0)
               + jnp.where((mrow == 1) & act, 1, 0))
        laste = jnp.where(nbi > 0, jnp.int32(e), laste)
        off = off + nb * _BM
    total = (off / _BM).astype(jnp.int32)
    blk = blk + jnp.where((mrow == 0) & (mcol >= total), laste, 0)
    pos_ref[...] = pos.astype(jnp.int32)
    meta_ref[...] = blk


def _scatter_body(pos_hbm, st_hbm, pos_v, st_v):
    c = lax.axis_index("c")
    s = lax.axis_index("s")

    @pl.when((c == 0) & (s == 0))
    def _():
        pltpu.sync_copy(pos_hbm, pos_v)

        def zero(j, carry):
            st_v[pl.ds(j * 16, 16)] = jnp.zeros((16,), jnp.int32)
            return carry

        lax.fori_loop(0, _NP // 16, zero, 0)
        iota = lax.iota(jnp.int32, 16)

        def scat(j, carry):
            idx = pos_v[pl.ds(j * 16, 16)]
            tok = lax.shift_right_logical(j * 16 + iota, 1)
            plsc.store_scatter(st_v, [idx], tok)
            return carry

        lax.fori_loop(0, _PAIRS // 16, scat, 0)
        pltpu.sync_copy(st_v, st_hbm)


def _gatherx_body(st_hbm, x_hbm, xs_hbm, idx_v, buf, gs0, gs1, gs2, ws0,
                  ws1, ws2):
    c = lax.axis_index("c")
    s = lax.axis_index("s")
    wid = s * 2 + c
    rows = _NP // _NSC
    base = wid * rows
    nch = rows // 16
    pltpu.sync_copy(st_hbm.at[pl.ds(base, rows)], idx_v)
    gsem = (gs0, gs1, gs2)
    wsem = (ws0, ws1, ws2)
    cg = [None] * nch
    cw = [None] * nch
    for j in range(nch):
        if j >= 3:
            cw[j - 3].wait()
        cg[j] = pltpu.async_copy(
            x_hbm.at[idx_v.at[pl.ds(j * 16, 16)]], buf.at[j % 3], gsem[j % 3])
        if j >= 2:
            cg[j - 2].wait()
            cw[j - 2] = pltpu.async_copy(
                buf.at[(j - 2) % 3], xs_hbm.at[pl.ds(base + (j - 2) * 16, 16)],
                wsem[(j - 2) % 3])
    for j in range(nch - 2, nch):
        cg[j].wait()
        cw[j] = pltpu.async_copy(
            buf.at[j % 3], xs_hbm.at[pl.ds(base + j * 16, 16)], wsem[j % 3])
    for j in range(nch - 3, nch):
        cw[j].wait()


def _gemm_body(meta_ref, xs_ref, w1_ref, w3_ref, w2_ref, o_ref):
    b = pl.program_id(0)
    valid = meta_ref[_NB + b]

    @pl.when(valid == 1)
    def _():
        xs = xs_ref[...]
        g = jnp.dot(xs, w1_ref[0], preferred_element_type=jnp.float32)
        u = jnp.dot(xs, w3_ref[0], preferred_element_type=jnp.float32)
        h = (g * jax.nn.sigmoid(g)) * u
        o_ref[...] = jnp.dot(h, w2_ref[0], preferred_element_type=jnp.float32)


def _combine_body(pos_hbm, w_hbm, o_hbm, out_hbm, pos_v, w_v, idxa_v, idxb_v,
                  bufa, bufb, outb, sema, semb):
    c = lax.axis_index("c")
    s = lax.axis_index("s")
    wid = s * 2 + c
    tok_per = _T // _NSC
    base_t = wid * tok_per
    pltpu.sync_copy(pos_hbm.at[pl.ds(2 * base_t, 2 * tok_per)], pos_v)
    pltpu.sync_copy(w_hbm.at[pl.ds(2 * base_t, 2 * tok_per)], w_v)
    iota = lax.iota(jnp.int32, 16)

    def chunk(cc, carry):
        lpair = cc * 32 + 2 * iota
        idxa_v[...] = plsc.load_gather(pos_v, [lpair])
        idxb_v[...] = plsc.load_gather(pos_v, [lpair + 1])
        cpa = pltpu.async_copy(o_hbm.at[idxa_v], bufa, sema)
        cpb = pltpu.async_copy(o_hbm.at[idxb_v], bufb, semb)
        cpa.wait()
        cpb.wait()

        def row(r, carry2):
            wa = plsc.load_gather(w_v, [cc * 32 + 2 * r + jnp.zeros((16,), jnp.int32)])
            wb = plsc.load_gather(w_v, [cc * 32 + 2 * r + 1 + jnp.zeros((16,), jnp.int32)])

            @plsc.parallel_loop(0, _H // 16, unroll=8)
            def vec(v):
                outb[r, pl.ds(v * 16, 16)] = (
                    wa * bufa[r, pl.ds(v * 16, 16)]
                    + wb * bufb[r, pl.ds(v * 16, 16)])

            return carry2

        lax.fori_loop(0, 16, row, 0)
        pltpu.sync_copy(outb, out_hbm.at[pl.ds(base_t + cc * 16, 16)])
        return carry

    lax.fori_loop(0, tok_per // 16, chunk, 0)


def kernel(hidden_states, W_gate, w1, w3, w2):
    orig_shape = hidden_states.shape
    x = hidden_states.reshape(-1, _H)
    t = x.shape[0]

    wg_pad = jnp.zeros((_H, _EPAD), jnp.float32).at[:, :_E].set(W_gate)

    bt_r = 512
    epair, wpair = pl.pallas_call(
        _router_body,
        grid=(t // bt_r,),
        in_specs=[
            pl.BlockSpec((bt_r, _H), lambda i: (i, 0)),
            pl.BlockSpec((_H, _EPAD), lambda i: (0, 0)),
        ],
        out_specs=[
            pl.BlockSpec((bt_r, _EPAD), lambda i: (i, 0)),
            pl.BlockSpec((bt_r, _EPAD), lambda i: (i, 0)),
        ],
        out_shape=[
            jax.ShapeDtypeStruct((t, _EPAD), jnp.int32),
            jax.ShapeDtypeStruct((t, _EPAD), jnp.float32),
        ],
    )(x, wg_pad)

    e_flat = epair[:, :2].reshape(_PAIRS)
    w_flat = wpair[:, :2].reshape(_PAIRS)

    pos32, meta = pl.pallas_call(
        _assign_body,
        in_specs=[pl.BlockSpec((32, 128), lambda: (0, 0))],
        out_specs=[
            pl.BlockSpec((32, 128), lambda: (0, 0)),
            pl.BlockSpec((8, 128), lambda: (0, 0)),
        ],
        out_shape=[
            jax.ShapeDtypeStruct((32, 128), jnp.int32),
            jax.ShapeDtypeStruct((8, 128), jnp.int32),
        ],
    )(e_flat.reshape(32, 128))

    pos_flat = pos32.reshape(_PAIRS)
    meta_vec = jnp.concatenate([meta[0, :_NB], meta[1, :_NB]])

    mesh = plsc.VectorSubcoreMesh(core_axis_name="c", subcore_axis_name="s")

    sorted_tok = pl.kernel(
        _scatter_body,
        out_type=jax.ShapeDtypeStruct((_NP,), jnp.int32),
        mesh=mesh,
        compiler_params=pltpu.CompilerParams(needs_layout_passes=False),
        scratch_types=[
            pltpu.VMEM((_PAIRS,), jnp.int32),
            pltpu.VMEM((_NP,), jnp.int32),
        ],
    )(pos_flat)

    x_sorted = pl.kernel(
        _gatherx_body,
        out_type=jax.ShapeDtypeStruct((_NP, _H), jnp.float32),
        mesh=mesh,
        scratch_types=[
            pltpu.VMEM((_NP // _NSC,), jnp.int32),
            pltpu.VMEM((3, 16, _H), jnp.float32),
            pltpu.SemaphoreType.DMA,
            pltpu.SemaphoreType.DMA,
            pltpu.SemaphoreType.DMA,
            pltpu.SemaphoreType.DMA,
            pltpu.SemaphoreType.DMA,
            pltpu.SemaphoreType.DMA,
        ],
    )(sorted_tok, x)

    o_sorted = pl.pallas_call(
        _gemm_body,
        grid_spec=pltpu.PrefetchScalarGridSpec(
            num_scalar_prefetch=1,
            grid=(_NB,),
            in_specs=[
                pl.BlockSpec((_BM, _H), lambda b, m: (b, 0)),
                pl.BlockSpec((1, _H, _I), lambda b, m: (m[b], 0, 0)),
                pl.BlockSpec((1, _H, _I), lambda b, m: (m[b], 0, 0)),
                pl.BlockSpec((1, _I, _H), lambda b, m: (m[b], 0, 0)),
            ],
            out_specs=pl.BlockSpec((_BM, _H), lambda b, m: (b, 0)),
        ),
        out_shape=jax.ShapeDtypeStruct((_NP, _H), jnp.float32),
        compiler_params=pltpu.CompilerParams(
            dimension_semantics=("arbitrary",)),
    )(meta_vec, x_sorted, w1, w3, w2)

    out = pl.kernel(
        _combine_body,
        out_type=jax.ShapeDtypeStruct((t, _H), jnp.float32),
        mesh=mesh,
        compiler_params=pltpu.CompilerParams(needs_layout_passes=False),
        scratch_types=[
            pltpu.VMEM((2 * _T // _NSC,), jnp.int32),
            pltpu.VMEM((2 * _T // _NSC,), jnp.float32),
            pltpu.VMEM((16,), jnp.int32),
            pltpu.VMEM((16,), jnp.int32),
            pltpu.VMEM((16, _H), jnp.float32),
            pltpu.VMEM((16, _H), jnp.float32),
            pltpu.VMEM((16, _H), jnp.float32),
            pltpu.SemaphoreType.DMA,
            pltpu.SemaphoreType.DMA,
        ],
    )(pos_flat, w_flat, o_sorted)

    return out.reshape(orig_shape)


# speedup vs baseline: 1.1220x; 1.0064x over previous
import functools

import jax
import jax.numpy as jnp
from jax import lax
from jax.experimental import pallas as pl
from jax.experimental.pallas import tpu as pltpu
from jax.experimental.pallas import tpu_sc as plsc

_E = 8
_H = 2048
_I = 1024
_EPAD = 128
_T = 2048
_PAIRS = 2 * _T
_BM = 256
_NP = _PAIRS + _E * _BM
_NB = _NP // _BM
_NSC = 32


def _router_body(x_ref, wg_ref, ep_ref, wp_ref):
    logits = jnp.dot(x_ref[...], wg_ref[...], preferred_element_type=jnp.float32)
    col = jax.lax.broadcasted_iota(jnp.int32, logits.shape, 1)
    masked = jnp.where(col < _E, logits, jnp.float32(-1e30))
    m = jnp.max(masked, axis=1, keepdims=True)
    p = jnp.exp(masked - m)
    p = p / jnp.sum(p, axis=1, keepdims=True)
    w1v = jnp.max(p, axis=1, keepdims=True)
    i1 = jnp.argmax(p, axis=1)[:, None]
    p2 = jnp.where(col == i1, jnp.float32(-1.0), p)
    w2v = jnp.max(p2, axis=1, keepdims=True)
    i2 = jnp.argmax(p2, axis=1)[:, None]
    s = w1v + w2v
    ep_ref[...] = (jnp.where(col == 0, i1, 0)
                   + jnp.where(col == 1, i2, 0)).astype(jnp.int32)
    wp_ref[...] = jnp.where(col == 0, w1v / s, 0.0) + jnp.where(
        col == 1, w2v / s, 0.0)


def _assign_body(e32_ref, pos_ref, meta_ref):
    ef = e32_ref[...]
    ri = jax.lax.broadcasted_iota(jnp.int32, (128, 128), 0)
    ci = jax.lax.broadcasted_iota(jnp.int32, (128, 128), 1)
    tri128 = (ri < ci).astype(jnp.float32)
    ones128 = jnp.ones((128, 128), jnp.float32)
    r32 = jax.lax.broadcasted_iota(jnp.int32, (32, 32), 0)
    c32 = jax.lax.broadcasted_iota(jnp.int32, (32, 32), 1)
    tri32 = (c32 < r32).astype(jnp.float32)
    mrow = jax.lax.broadcasted_iota(jnp.int32, (8, 128), 0)
    mcol = jax.lax.broadcasted_iota(jnp.int32, (8, 128), 1)

    pos = jnp.zeros((32, 128), jnp.float32)
    blk = jnp.zeros((8, 128), jnp.int32)
    off = jnp.float32(0.0)
    laste = jnp.int32(0)
    for e in range(_E):
        mask = (ef == e).astype(jnp.float32)
        intra = jnp.dot(mask, tri128, preferred_element_type=jnp.float32)
        srow_b = jnp.dot(mask, ones128, preferred_element_type=jnp.float32)
        rowpref = jnp.dot(tri32, srow_b, preferred_element_type=jnp.float32)
        rank = intra + rowpref
        cnt = jnp.sum(mask)
        nb = jnp.ceil(cnt / _BM)
        pos = pos + mask * (off + rank)
        start = (off / _BM).astype(jnp.int32)
        nbi = nb.astype(jnp.int32)
        act = (mcol >= start) & (mcol < start + nbi)
        blk = (blk + jnp.where((mrow == 0) & act, e, 0)
               + jnp.where((mrow == 1) & act, 1, 0))
        laste = jnp.where(nbi > 0, jnp.int32(e), laste)
        off = off + nb * _BM
    total = (off / _BM).astype(jnp.int32)
    blk = blk + jnp.where((mrow == 0) & (mcol >= total), laste, 0)
    pos_ref[...] = pos.astype(jnp.int32)
    meta_ref[...] = blk


def _scatter_body(pos_hbm, st_hbm, pos_v, st_v):
    c = lax.axis_index("c")
    s = lax.axis_index("s")

    @pl.when((c == 0) & (s == 0))
    def _():
        pltpu.sync_copy(pos_hbm, pos_v)

        def zero(j, carry):
            st_v[pl.ds(j * 16, 16)] = jnp.zeros((16,), jnp.int32)
            return carry

        lax.fori_loop(0, _NP // 16, zero, 0)
        iota = lax.iota(jnp.int32, 16)

        def scat(j, carry):
            idx = pos_v[pl.ds(j * 16, 16)]
            tok = lax.shift_right_logical(j * 16 + iota, 1)
            plsc.store_scatter(st_v, [idx], tok)
            return carry

        lax.fori_loop(0, _PAIRS // 16, scat, 0)
        pltpu.sync_copy(st_v, st_hbm)


def _gatherx_body(st_hbm, x_hbm, xs_hbm, idx2, buf2, gs0, gs1):
    c = lax.axis_index("c")
    s = lax.axis_index("s")
    wid = s * 2 + c
    rows = _NP // _NSC
    base = wid * rows
    ch = 24
    nch = rows // ch
    gsem = (gs0, gs1)
    cps = [None] * nch
    pltpu.sync_copy(st_hbm.at[pl.ds(base, ch)], idx2.at[0])
    cps[0] = pltpu.async_copy(x_hbm.at[idx2.at[0]], buf2.at[0], gsem[0])
    for j in range(nch):
        if j < nch - 1:
            k = (j + 1) % 2
            pltpu.sync_copy(st_hbm.at[pl.ds(base + (j + 1) * ch, ch)],
                            idx2.at[k])
            cps[j + 1] = pltpu.async_copy(x_hbm.at[idx2.at[k]], buf2.at[k],
                                          gsem[k])
        cps[j].wait()
        pltpu.sync_copy(buf2.at[j % 2], xs_hbm.at[pl.ds(base + j * ch, ch)])


def _gemm_body(meta_ref, xs_ref, w1_ref, w3_ref, w2_ref, o_ref):
    b = pl.program_id(0)
    valid = meta_ref[_NB + b]

    @pl.when(valid == 1)
    def _():
        xs = xs_ref[...]
        g = jnp.dot(xs, w1_ref[0], preferred_element_type=jnp.float32)
        u = jnp.dot(xs, w3_ref[0], preferred_element_type=jnp.float32)
        h = (g * jax.nn.sigmoid(g)) * u
        o_ref[...] = jnp.dot(h, w2_ref[0], preferred_element_type=jnp.float32)


def _combine_body(pos_hbm, w_hbm, o_hbm, out_hbm, pos_v, w_v, idxa_v, idxb_v,
                  bufa, bufb, outb, sema, semb):
    c = lax.axis_index("c")
    s = lax.axis_index("s")
    wid = s * 2 + c
    tok_per = _T // _NSC
    base_t = wid * tok_per
    pltpu.sync_copy(pos_hbm.at[pl.ds(2 * base_t, 2 * tok_per)], pos_v)
    pltpu.sync_copy(w_hbm.at[pl.ds(2 * base_t, 2 * tok_per)], w_v)
    iota = lax.iota(jnp.int32, 16)

    def chunk(cc, carry):
        lpair = cc * 32 + 2 * iota
        idxa_v[...] = plsc.load_gather(pos_v, [lpair])
        idxb_v[...] = plsc.load_gather(pos_v, [lpair + 1])
        cpa = pltpu.async_copy(o_hbm.at[idxa_v], bufa, sema)
        cpb = pltpu.async_copy(o_hbm.at[idxb_v], bufb, semb)
        cpa.wait()
        cpb.wait()

        def row(r, carry2):
            wa = plsc.load_gather(w_v, [cc * 32 + 2 * r + jnp.zeros((16,), jnp.int32)])
            wb = plsc.load_gather(w_v, [cc * 32 + 2 * r + 1 + jnp.zeros((16,), jnp.int32)])

            @plsc.parallel_loop(0, _H // 16, unroll=8)
            def vec(v):
                outb[r, pl.ds(v * 16, 16)] = (
                    wa * bufa[r, pl.ds(v * 16, 16)]
                    + wb * bufb[r, pl.ds(v * 16, 16)])

            return carry2

        lax.fori_loop(0, 16, row, 0)
        pltpu.sync_copy(outb, out_hbm.at[pl.ds(base_t + cc * 16, 16)])
        return carry

    lax.fori_loop(0, tok_per // 16, chunk, 0)


def kernel(hidden_states, W_gate, w1, w3, w2):
    orig_shape = hidden_states.shape
    x = hidden_states.reshape(-1, _H)
    t = x.shape[0]

    wg_pad = jnp.zeros((_H, _EPAD), jnp.float32).at[:, :_E].set(W_gate)

    bt_r = 512
    epair, wpair = pl.pallas_call(
        _router_body,
        grid=(t // bt_r,),
        in_specs=[
            pl.BlockSpec((bt_r, _H), lambda i: (i, 0)),
            pl.BlockSpec((_H, _EPAD), lambda i: (0, 0)),
        ],
        out_specs=[
            pl.BlockSpec((bt_r, _EPAD), lambda i: (i, 0)),
            pl.BlockSpec((bt_r, _EPAD), lambda i: (i, 0)),
        ],
        out_shape=[
            jax.ShapeDtypeStruct((t, _EPAD), jnp.int32),
            jax.ShapeDtypeStruct((t, _EPAD), jnp.float32),
        ],
    )(x, wg_pad)

    e_flat = epair[:, :2].reshape(_PAIRS)
    w_flat = wpair[:, :2].reshape(_PAIRS)

    pos32, meta = pl.pallas_call(
        _assign_body,
        in_specs=[pl.BlockSpec((32, 128), lambda: (0, 0))],
        out_specs=[
            pl.BlockSpec((32, 128), lambda: (0, 0)),
            pl.BlockSpec((8, 128), lambda: (0, 0)),
        ],
        out_shape=[
            jax.ShapeDtypeStruct((32, 128), jnp.int32),
            jax.ShapeDtypeStruct((8, 128), jnp.int32),
        ],
    )(e_flat.reshape(32, 128))

    pos_flat = pos32.reshape(_PAIRS)
    meta_vec = jnp.concatenate([meta[0, :_NB], meta[1, :_NB]])

    mesh = plsc.VectorSubcoreMesh(core_axis_name="c", subcore_axis_name="s")

    sorted_tok = pl.kernel(
        _scatter_body,
        out_type=jax.ShapeDtypeStruct((_NP,), jnp.int32),
        mesh=mesh,
        compiler_params=pltpu.CompilerParams(needs_layout_passes=False),
        scratch_types=[
            pltpu.VMEM((_PAIRS,), jnp.int32),
            pltpu.VMEM((_NP,), jnp.int32),
        ],
    )(pos_flat)

    x_sorted = pl.kernel(
        _gatherx_body,
        out_type=jax.ShapeDtypeStruct((_NP, _H), jnp.float32),
        mesh=mesh,
        scratch_types=[
            pltpu.VMEM((2, 24), jnp.int32),
            pltpu.VMEM((2, 24, _H), jnp.float32),
            pltpu.SemaphoreType.DMA,
            pltpu.SemaphoreType.DMA,
        ],
    )(sorted_tok, x)

    o_sorted = pl.pallas_call(
        _gemm_body,
        grid_spec=pltpu.PrefetchScalarGridSpec(
            num_scalar_prefetch=1,
            grid=(_NB,),
            in_specs=[
                pl.BlockSpec((_BM, _H), lambda b, m: (b, 0)),
                pl.BlockSpec((1, _H, _I), lambda b, m: (m[b], 0, 0)),
                pl.BlockSpec((1, _H, _I), lambda b, m: (m[b], 0, 0)),
                pl.BlockSpec((1, _I, _H), lambda b, m: (m[b], 0, 0)),
            ],
            out_specs=pl.BlockSpec((_BM, _H), lambda b, m: (b, 0)),
        ),
        out_shape=jax.ShapeDtypeStruct((_NP, _H), jnp.float32),
        compiler_params=pltpu.CompilerParams(
            dimension_semantics=("arbitrary",)),
    )(meta_vec, x_sorted, w1, w3, w2)

    out = pl.kernel(
        _combine_body,
        out_type=jax.ShapeDtypeStruct((t, _H), jnp.float32),
        mesh=mesh,
        compiler_params=pltpu.CompilerParams(needs_layout_passes=False),
        scratch_types=[
            pltpu.VMEM((2 * _T // _NSC,), jnp.int32),
            pltpu.VMEM((2 * _T // _NSC,), jnp.float32),
            pltpu.VMEM((16,), jnp.int32),
            pltpu.VMEM((16,), jnp.int32),
            pltpu.VMEM((16, _H), jnp.float32),
            pltpu.VMEM((16, _H), jnp.float32),
            pltpu.VMEM((16, _H), jnp.float32),
            pltpu.SemaphoreType.DMA,
            pltpu.SemaphoreType.DMA,
        ],
    )(pos_flat, w_flat, o_sorted)

    return out.reshape(orig_shape)
